# baseline (speedup 1.0000x reference)
import functools

import jax
import jax.numpy as jnp
from jax import lax
from jax.experimental import pallas as pl
from jax.experimental.pallas import tpu as pltpu
from jax.experimental.pallas import tpu_sc as plsc

VOCAB_D = 64
NC, NS = 2, 16
NW = NC * NS
CHUNK = 512


@functools.partial(jax.jit, static_argnames=("b_total",))
def _gather_rows(table, idx_flat, b_total):
    per_w = b_total // NW
    n_chunks = per_w // CHUNK
    mesh = plsc.VectorSubcoreMesh(core_axis_name="c", subcore_axis_name="s")

    @functools.partial(
        pl.kernel,
        mesh=mesh,
        compiler_params=pltpu.CompilerParams(use_tc_tiling_on_sc=False),
        out_type=jax.ShapeDtypeStruct((b_total, VOCAB_D), jnp.float32),
        scratch_types=[
            pltpu.VMEM((CHUNK,), jnp.int32),
            pltpu.VMEM((CHUNK, VOCAB_D), jnp.float32),
            pltpu.SemaphoreType.DMA,
        ],
    )
    def k(table_hbm, idx_hbm, out_hbm, idx_v, rows_v, sem):
        wid = lax.axis_index("s") * NC + lax.axis_index("c")
        base0 = wid * per_w

        @pl.loop(0, n_chunks)
        def _chunk(i):
            base = base0 + i * CHUNK
            pltpu.sync_copy(idx_hbm.at[pl.ds(base, CHUNK)], idx_v)
            pltpu.async_copy(table_hbm.at[idx_v], rows_v, sem).wait()
            pltpu.sync_copy(rows_v, out_hbm.at[pl.ds(base, CHUNK)])

    return k(table, idx_flat)


def kernel(tokens, table):
    b, s = tokens.shape
    idx_flat = jnp.reshape(tokens, (b * s,)).astype(jnp.int32)
    out = _gather_rows(table, idx_flat, b * s)
    return jnp.reshape(out, (b, s, VOCAB_D))


# speedup vs baseline: 1.0471x; 1.0471x over previous
import functools

import jax
import jax.numpy as jnp
from jax import lax
from jax.experimental import pallas as pl
from jax.experimental.pallas import tpu as pltpu
from jax.experimental.pallas import tpu_sc as plsc

D = 64
NC, NS = 2, 16
NW = NC * NS
CHUNK = 640


@functools.partial(jax.jit, static_argnames=("b_total",))
def _gather_rows(table, idx_grouped, b_total):
    per_w = b_total // NW
    n_chunks = per_w // CHUNK
    mesh = plsc.VectorSubcoreMesh(core_axis_name="c", subcore_axis_name="s")

    @functools.partial(
        pl.kernel,
        mesh=mesh,
        compiler_params=pltpu.CompilerParams(use_tc_tiling_on_sc=False),
        out_type=jax.ShapeDtypeStruct((b_total, D), jnp.float32),
        scratch_types=[
            pltpu.VMEM((n_chunks, CHUNK), jnp.int32),
            pltpu.VMEM((2, CHUNK, D), jnp.float32),
            pltpu.SemaphoreType.DMA,
            pltpu.SemaphoreType.DMA,
            pltpu.SemaphoreType.DMA,
            pltpu.SemaphoreType.DMA,
        ],
    )
    def k(table_hbm, idx_hbm, out_hbm, idx_v, rows_v, g0, g1, s0, s1):
        wid = lax.axis_index("s") * NC + lax.axis_index("c")
        base0 = wid * per_w
        gsem = (g0, g1)
        ssem = (s0, s1)

        pltpu.sync_copy(idx_hbm.at[wid], idx_v)

        def start_gather(j, b):
            pltpu.async_copy(table_hbm.at[idx_v.at[j]], rows_v.at[b], gsem[b])

        def wait_gather(j, b):
            pltpu.make_async_copy(
                table_hbm.at[idx_v.at[j]], rows_v.at[b], gsem[b]
            ).wait()

        def start_store(j, b):
            pltpu.async_copy(
                rows_v.at[b], out_hbm.at[pl.ds(base0 + j * CHUNK, CHUNK)], ssem[b]
            )

        def wait_store(j, b):
            pltpu.make_async_copy(
                rows_v.at[b], out_hbm.at[pl.ds(base0 + j * CHUNK, CHUNK)], ssem[b]
            ).wait()

        start_gather(0, 0)

        @pl.loop(0, n_chunks, step=2)
        def _group(i):
            for b in range(2):
                j = i + b
                wait_gather(j, b)

                @pl.when(j + 1 < n_chunks)
                def _():
                    @pl.when(j >= 1)
                    def _():
                        wait_store(j - 1, 1 - b)

                    start_gather(j + 1, 1 - b)

                start_store(j, b)

        wait_store(n_chunks - 2, 0)
        wait_store(n_chunks - 1, 1)

    return k(table, idx_grouped)


def kernel(tokens, table):
    b, s = tokens.shape
    n = b * s
    idx_grouped = jnp.reshape(tokens.astype(jnp.int32), (NW, (n // NW) // CHUNK, CHUNK))
    out = _gather_rows(table, idx_grouped, n)
    return jnp.reshape(out, (b, s, D))
